# baseline (speedup 1.0000x reference)
import functools

import jax
import jax.numpy as jnp
from jax import lax
from jax.experimental import pallas as pl
from jax.experimental.pallas import tpu as pltpu
from jax.experimental.pallas import tpu_sc as plsc

DIM = 64
NC = 2
NS = 16
BBLK = 2048
QUART = 4096
LB = 16


def _gather_t(pe_t, pos_t, seq, nbatch, nrows):
    mesh = plsc.VectorSubcoreMesh(core_axis_name="c", subcore_axis_name="s")
    nbank = (seq + LB - 1) // LB

    @functools.partial(
        pl.kernel,
        mesh=mesh,
        out_type=jax.ShapeDtypeStruct((seq, DIM, nbatch), jnp.float32),
        scratch_types=[
            pltpu.VMEM((8, nrows), jnp.float32),
            pltpu.VMEM((QUART,), jnp.int32),
            pltpu.VMEM((QUART,), jnp.int32),
            pltpu.VMEM((8, BBLK), jnp.float32),
            pltpu.VMEM((8, BBLK), jnp.float32),
            pltpu.VMEM_SHARED((2, LB, 2 * QUART), jnp.int32),
            pltpu.SemaphoreType.DMA,
            pltpu.SemaphoreType.DMA,
            pltpu.SemaphoreType.DMA,
            pltpu.SemaphoreType.DMA,
            pltpu.SemaphoreType.DMA,
        ],
        compiler_params=pltpu.CompilerParams(
            use_tc_tiling_on_sc=True, needs_layout_passes=False),
    )
    def k(pe_hbm, pos_hbm, out_hbm, pe_v, idx0, idx1, wb0, wb1, sp,
          isem0, isem1, osem0, osem1, ssem):
        c = lax.axis_index("c")
        s = lax.axis_index("s")
        octet = lax.rem(s, 8)
        quarter = c * 2 + s // 8
        d0 = octet * 8
        bq = quarter * QUART
        qloc = (s // 8) * QUART
        half = c * 2 * QUART
        idx_v = (idx0, idx1)
        isem = (isem0, isem1)
        wb = (wb0, wb1)
        osem = (osem0, osem1)

        def stage_row(l2, bank):
            return pltpu.make_async_copy(
                pos_hbm.at[l2, pl.ds(half, 2 * QUART)], sp.at[bank, s], ssem)

        def fetch_idx(bank, li, p):
            return pltpu.make_async_copy(
                sp.at[bank, li, pl.ds(qloc, QUART)], idx_v[p], isem[p])

        pltpu.sync_copy(pe_hbm.at[pl.ds(d0, 8), :], pe_v)
        stage_row(s, 0).start()
        stage_row(s, 0).wait()
        plsc.subcore_barrier()
        fetch_idx(0, 0, 0).start()

        def gather_block(src_idx, boff, dst):
            @plsc.parallel_loop(0, BBLK // 16, 1, unroll=4)
            def g_body(g):
                i16 = src_idx[pl.ds(boff + g * 16, 16)]
                for qd in range(8):
                    rows = jnp.full((16,), qd, jnp.int32)
                    dst[qd, pl.ds(g * 16, 16)] = plsc.load_gather(
                        pe_v, [rows, i16])

        def row_body(lb, li, bank, pp):
            l = lb * LB + li
            fetch_idx(bank, 0, pp).wait()

            @pl.when((li + 1 < LB) & (l + 1 < seq))
            def _():
                fetch_idx(bank, li + 1, 1 - pp).start()

            for bb in range(2):
                @pl.when(l >= 1)
                def _():
                    pltpu.make_async_copy(
                        wb[bb],
                        out_hbm.at[0, pl.ds(d0, 8),
                                   pl.ds(bq + bb * BBLK, BBLK)],
                        osem[bb]).wait()
                gather_block(idx_v[pp], bb * BBLK, wb[bb])
                pltpu.async_copy(
                    wb[bb],
                    out_hbm.at[l, pl.ds(d0, 8), pl.ds(bq + bb * BBLK, BBLK)],
                    osem[bb])

        def bank_body(lb, carry):
            bank = lax.rem(lb, 2)
            l2 = (lb + 1) * LB + s
            stage_next = (lb + 1 < nbank) & (l2 < seq)

            for bk in range(2):
                @pl.when(bank == bk)
                def _():
                    @pl.when(stage_next)
                    def _():
                        stage_row(l2, 1 - bk).start()

                    def li_body(li, carry2):
                        for pp in range(2):
                            @pl.when(lax.rem(li, 2) == pp)
                            def _():
                                @pl.when(lb * LB + li < seq)
                                def _():
                                    row_body(lb, li, bk, pp)
                        return carry2

                    lax.fori_loop(0, LB, li_body, 0)

                    @pl.when(stage_next)
                    def _():
                        stage_row(l2, 1 - bk).wait()
                    plsc.subcore_barrier()

                    @pl.when(lb + 1 < nbank)
                    def _():
                        fetch_idx(1 - bk, 0, 0).start()
            return carry

        lax.fori_loop(0, nbank, bank_body, 0)
        for bb in range(2):
            pltpu.make_async_copy(
                wb[bb],
                out_hbm.at[0, pl.ds(d0, 8), pl.ds(bq + bb * BBLK, BBLK)],
                osem[bb]).wait()

    return k(pe_t, pos_t)


def kernel(pos, pe):
    b, l = pos.shape
    pos_t = pos.T.astype(jnp.int32)
    pe_t = pe.T
    w = _gather_t(pe_t, pos_t, l, b, pe.shape[0])
    return jnp.transpose(w, (2, 0, 1))


# speedup vs baseline: 1.1005x; 1.1005x over previous
import functools

import jax
import jax.numpy as jnp
from jax import lax
from jax.experimental import pallas as pl
from jax.experimental.pallas import tpu as pltpu
from jax.experimental.pallas import tpu_sc as plsc

DIM = 64
NC = 2
NS = 16
BBLK = 2048
QUART = 4096


def _gather_t(pe_t, pos_t, seq, nbatch, nrows):
    mesh = plsc.VectorSubcoreMesh(core_axis_name="c", subcore_axis_name="s")

    @functools.partial(
        pl.kernel,
        mesh=mesh,
        out_type=jax.ShapeDtypeStruct((seq, DIM, nbatch), jnp.float32),
        scratch_types=[
            pltpu.VMEM((8, nrows), jnp.float32),
            pltpu.VMEM((QUART,), jnp.int32),
            pltpu.VMEM((QUART,), jnp.int32),
            pltpu.VMEM((8, BBLK), jnp.float32),
            pltpu.VMEM((8, BBLK), jnp.float32),
            pltpu.SemaphoreType.DMA,
            pltpu.SemaphoreType.DMA,
            pltpu.SemaphoreType.DMA,
            pltpu.SemaphoreType.DMA,
        ],
        compiler_params=pltpu.CompilerParams(
            use_tc_tiling_on_sc=True, needs_layout_passes=False),
    )
    def k(pe_hbm, pos_hbm, out_hbm, pe_v, idx0, idx1, wb0, wb1,
          isem0, isem1, osem0, osem1):
        c = lax.axis_index("c")
        s = lax.axis_index("s")
        octet = lax.rem(s, 8)
        quarter = c * 2 + s // 8
        d0 = octet * 8
        bq = quarter * QUART
        idx_v = (idx0, idx1)
        isem = (isem0, isem1)
        wb = (wb0, wb1)
        osem = (osem0, osem1)

        pltpu.sync_copy(pe_hbm.at[pl.ds(d0, 8), :], pe_v)
        pltpu.async_copy(pos_hbm.at[0, pl.ds(bq, QUART)], idx0, isem0)

        def gather_block(src_idx, boff, dst):
            @plsc.parallel_loop(0, BBLK // 16, 1, unroll=4)
            def g_body(g):
                i16 = src_idx[pl.ds(boff + g * 16, 16)]
                for qd in range(8):
                    rows = jnp.full((16,), qd, jnp.int32)
                    dst[qd, pl.ds(g * 16, 16)] = plsc.load_gather(
                        pe_v, [rows, i16])

        def body(l, carry):
            p = lax.rem(l, 2)
            for pp in range(2):
                @pl.when(p == pp)
                def _():
                    pltpu.make_async_copy(
                        pos_hbm.at[0, pl.ds(bq, QUART)],
                        idx_v[pp], isem[pp]).wait()

                    @pl.when(l + 1 < seq)
                    def _():
                        pltpu.async_copy(
                            pos_hbm.at[l + 1, pl.ds(bq, QUART)],
                            idx_v[1 - pp], isem[1 - pp])

                    for bb in range(2):
                        @pl.when(l >= 1)
                        def _():
                            pltpu.make_async_copy(
                                wb[bb],
                                out_hbm.at[0, pl.ds(d0, 8),
                                           pl.ds(bq + bb * BBLK, BBLK)],
                                osem[bb]).wait()
                        gather_block(idx_v[pp], bb * BBLK, wb[bb])
                        pltpu.async_copy(
                            wb[bb],
                            out_hbm.at[l, pl.ds(d0, 8),
                                       pl.ds(bq + bb * BBLK, BBLK)],
                            osem[bb])
            return carry

        lax.fori_loop(0, seq, body, 0)
        for bb in range(2):
            pltpu.make_async_copy(
                wb[bb],
                out_hbm.at[0, pl.ds(d0, 8), pl.ds(bq + bb * BBLK, BBLK)],
                osem[bb]).wait()

    return k(pe_t, pos_t)


def kernel(pos, pe):
    b, l = pos.shape
    pos_t = pos.T.astype(jnp.int32)
    pe_t = pe.T
    w = _gather_t(pe_t, pos_t, l, b, pe.shape[0])
    return jnp.transpose(w, (2, 0, 1))
